# baseline (speedup 1.0000x reference)
import functools

import jax
import jax.numpy as jnp
from jax import lax
from jax.experimental import pallas as pl
from jax.experimental.pallas import tpu as pltpu
from jax.experimental.pallas import tpu_sc as plsc

N_SENSORS = 2000
D_IN = 16
ROWS_PER_WORKER = 80
ACTIVE_WORKERS = N_SENSORS // ROWS_PER_WORKER


def _sc_gather(table_hbm, idx_hbm, out_hbm, idx_v, rows_v, sem):
    wid = lax.axis_index("s") * 2 + lax.axis_index("c")

    @pl.when(wid < ACTIVE_WORKERS)
    def _():
        base = wid * ROWS_PER_WORKER
        pltpu.sync_copy(idx_hbm.at[pl.ds(base, ROWS_PER_WORKER)], idx_v)
        pltpu.async_copy(table_hbm.at[idx_v], rows_v, sem).wait()
        pltpu.sync_copy(rows_v, out_hbm.at[pl.ds(base, ROWS_PER_WORKER)])


def _mlp_body(x_ref, w1_ref, b1_ref, w2_ref, b2_ref, o_ref):
    x = x_ref[...]
    h = jnp.dot(x, w1_ref[...], preferred_element_type=jnp.float32)
    h = jnp.maximum(h + b1_ref[...], 0.0)
    y = jnp.dot(h, w2_ref[...], preferred_element_type=jnp.float32)
    o_ref[...] = jnp.maximum(y + b2_ref[...], 0.0)


def kernel(edge_attribute, adj_row, adj_col, adj_val, sensor_indexes, W1, b1, W2, b2):
    del adj_row, adj_col, adj_val

    gather = functools.partial(
        pl.kernel,
        out_type=jax.ShapeDtypeStruct((N_SENSORS, D_IN), jnp.float32),
        mesh=plsc.VectorSubcoreMesh(core_axis_name="c", subcore_axis_name="s"),
        scratch_types=[
            pltpu.VMEM((ROWS_PER_WORKER,), jnp.int32),
            pltpu.VMEM((ROWS_PER_WORKER, D_IN), jnp.float32),
            pltpu.SemaphoreType.DMA,
        ],
        compiler_params=pltpu.CompilerParams(use_tc_tiling_on_sc=False),
    )(_sc_gather)

    gathered = gather(edge_attribute, sensor_indexes)

    out = pl.pallas_call(
        _mlp_body,
        out_shape=jax.ShapeDtypeStruct((N_SENSORS, W2.shape[1]), jnp.float32),
    )(gathered, W1, b1.reshape(1, -1), W2, b2.reshape(1, -1))
    return out


# speedup vs baseline: 1.0045x; 1.0045x over previous
import functools

import jax
import jax.numpy as jnp
from jax import lax
from jax.experimental import pallas as pl
from jax.experimental.pallas import tpu as pltpu
from jax.experimental.pallas import tpu_sc as plsc

N_SENSORS = 2000
D_IN = 16
WIDE = 128
PACK = WIDE // D_IN
ROWS_PER_WORKER = 80
ACTIVE_WORKERS = N_SENSORS // ROWS_PER_WORKER
LANES = 16


def _sc_gather(table_hbm, idx_hbm, out_hbm, idx_v, idx8_v, rows_v, sem):
    wid = lax.axis_index("s") * 2 + lax.axis_index("c")

    @pl.when(wid < ACTIVE_WORKERS)
    def _():
        base = wid * ROWS_PER_WORKER
        pltpu.sync_copy(idx_hbm.at[pl.ds(base, ROWS_PER_WORKER)], idx_v)
        for j in range(ROWS_PER_WORKER // LANES):
            sl = pl.ds(j * LANES, LANES)
            idx8_v[sl] = idx_v[sl] >> 3
        pltpu.async_copy(table_hbm.at[idx8_v], rows_v, sem).wait()
        pltpu.sync_copy(rows_v, out_hbm.at[pl.ds(base, ROWS_PER_WORKER)])


def _mlp_body(x_ref, idx_ref, w1_ref, b1_ref, w2_ref, b2_ref, o_ref):
    rem = idx_ref[...] & (PACK - 1)
    lane = lax.broadcasted_iota(jnp.int32, (N_SENSORS, WIDE), 1) >> 4
    xm = jnp.where(lane == rem, x_ref[...], 0.0)
    h = jnp.dot(xm, w1_ref[...], preferred_element_type=jnp.float32)
    h = jnp.maximum(h + b1_ref[...], 0.0)
    y = jnp.dot(h, w2_ref[...], preferred_element_type=jnp.float32)
    o_ref[...] = jnp.maximum(y + b2_ref[...], 0.0)


def kernel(edge_attribute, adj_row, adj_col, adj_val, sensor_indexes, W1, b1, W2, b2):
    del adj_row, adj_col, adj_val

    table_wide = edge_attribute.reshape(-1, WIDE)

    gather = functools.partial(
        pl.kernel,
        out_type=jax.ShapeDtypeStruct((N_SENSORS, WIDE), jnp.float32),
        mesh=plsc.VectorSubcoreMesh(core_axis_name="c", subcore_axis_name="s"),
        scratch_types=[
            pltpu.VMEM((ROWS_PER_WORKER,), jnp.int32),
            pltpu.VMEM((ROWS_PER_WORKER,), jnp.int32),
            pltpu.VMEM((ROWS_PER_WORKER, WIDE), jnp.float32),
            pltpu.SemaphoreType.DMA,
        ],
    )(_sc_gather)

    gathered = gather(table_wide, sensor_indexes)

    out = pl.pallas_call(
        _mlp_body,
        out_shape=jax.ShapeDtypeStruct((N_SENSORS, W2.shape[1]), jnp.float32),
    )(gathered, sensor_indexes.reshape(-1, 1), jnp.tile(W1, (PACK, 1)),
      b1.reshape(1, -1), W2, b2.reshape(1, -1))
    return out


# speedup vs baseline: 1.8677x; 1.8594x over previous
import functools

import jax
import jax.numpy as jnp
from jax import lax
from jax.experimental import pallas as pl
from jax.experimental.pallas import tpu as pltpu
from jax.experimental.pallas import tpu_sc as plsc

N_SENSORS = 2000
D_IN = 16
LANES = 16
N_WORKERS = 32
SENSORS_PER_WORKER = 64
CHUNK = 16


def _sc_gather(tT_hbm, idx_hbm, out_hbm, idxc_v, blkbuf, outv, sem):
    wid = lax.axis_index("s") * 2 + lax.axis_index("c")
    base = wid * SENSORS_PER_WORKER
    rows = lax.iota(jnp.int32, LANES)
    for chunk in range(SENSORS_PER_WORKER // CHUNK):
        cbase = base + chunk * CHUNK

        @pl.when(cbase < N_SENSORS)
        def _(cbase=cbase):
            pltpu.sync_copy(idx_hbm.at[pl.ds(cbase, CHUNK)], idxc_v)
            vec = idxc_v[...]
            blk = vec >> 7
            rem = vec & 127
            copies = []
            for k in range(CHUNK):
                off = pl.multiple_of(blk[k] * 128, 128)
                copies.append(
                    pltpu.async_copy(
                        tT_hbm.at[:, pl.ds(off, 128)],
                        blkbuf.at[:, pl.ds(k * 128, 128)],
                        sem,
                    )
                )
            for c in copies:
                c.wait()
            for k in range(CHUNK):
                cols = jnp.full((LANES,), k * 128, jnp.int32) + rem[k]
                outv[k] = plsc.load_gather(blkbuf, [rows, cols])
            pltpu.sync_copy(outv, out_hbm.at[pl.ds(cbase, CHUNK), :])


def _mlp_body(x_ref, w1_ref, b1_ref, w2_ref, b2_ref, o_ref):
    x = x_ref[...]
    h = jnp.dot(x, w1_ref[...], preferred_element_type=jnp.float32)
    h = jnp.maximum(h + b1_ref[...], 0.0)
    y = jnp.dot(h, w2_ref[...], preferred_element_type=jnp.float32)
    o_ref[...] = jnp.maximum(y + b2_ref[...], 0.0)


def kernel(edge_attribute, adj_row, adj_col, adj_val, sensor_indexes, W1, b1, W2, b2):
    del adj_row, adj_col, adj_val

    gather = functools.partial(
        pl.kernel,
        out_type=jax.ShapeDtypeStruct((N_SENSORS, D_IN), jnp.float32),
        mesh=plsc.VectorSubcoreMesh(core_axis_name="c", subcore_axis_name="s"),
        scratch_types=[
            pltpu.VMEM((CHUNK,), jnp.int32),
            pltpu.VMEM((LANES, CHUNK * 128), jnp.float32),
            pltpu.VMEM((CHUNK, D_IN), jnp.float32),
            pltpu.SemaphoreType.DMA,
        ],
        compiler_params=pltpu.CompilerParams(
            use_tc_tiling_on_sc=True, needs_layout_passes=False
        ),
    )(_sc_gather)

    gathered = gather(edge_attribute.T, sensor_indexes)

    out = pl.pallas_call(
        _mlp_body,
        out_shape=jax.ShapeDtypeStruct((N_SENSORS, W2.shape[1]), jnp.float32),
    )(gathered, W1, b1.reshape(1, -1), W2, b2.reshape(1, -1))
    return out
